# baseline (speedup 1.0000x reference)
import functools

import numpy as np
import jax
import jax.numpy as jnp
from jax import lax
from jax.experimental import pallas as pl
from jax.experimental.pallas import tpu as pltpu
from jax.experimental.pallas import tpu_sc as plsc

L = 16
F = 2
T = 2 ** 19
BASE = 16
MAXR = 4096
GEO = 15
_b = np.exp((np.log(MAXR) - np.log(BASE)) / (L - 1))
RES = [int(np.floor(BASE * _b ** l)) for l in range(L)]
P1 = int(np.uint32(2654435761).astype(np.int32))
P2 = int(np.uint32(805459861).astype(np.int32))
HMASK = T - 1

NC = 2
NS = 16
LANES = 16
NW = NC * NS

CH = 256
GP = CH // LANES


def _encode(pos_x, pos_y, pos_z, tab):
    n = pos_x.shape[0]
    n_per_w = n // NW
    n_chunks = n_per_w // CH

    n_groups = n_per_w // LANES
    mesh = plsc.VectorSubcoreMesh(core_axis_name="c", subcore_axis_name="s")

    @functools.partial(
        pl.kernel,
        out_type=jax.ShapeDtypeStruct((L * F, n), jnp.float32),
        mesh=mesh,
        scratch_types=[
            pltpu.VMEM((n_per_w,), jnp.float32),
            pltpu.VMEM((n_per_w,), jnp.float32),
            pltpu.VMEM((n_per_w,), jnp.float32),
            pltpu.VMEM((2 * L * 8 * LANES,), jnp.int32),
            pltpu.VMEM((2 * L * 8 * LANES,), jnp.int32),
            pltpu.VMEM((2, L, 8 * LANES), jnp.float32),
            pltpu.VMEM((2 * L * 8 * LANES,), jnp.float32),
            pltpu.VMEM((2 * L * 8 * LANES,), jnp.float32),
            pltpu.VMEM((L * F, CH), jnp.float32),
            pltpu.SemaphoreType.DMA,
        ],
    )
    def enc(px_hbm, py_hbm, pz_hbm, tab_hbm, out_hbm, xb, yb, zb, idx0,
            idx1, wb, gb0, gb1, ov, sem):
        idxs = (idx0, idx1)
        gbs = (gb0, gb1)
        wid = lax.axis_index("s") * NC + lax.axis_index("c")
        base = wid * n_per_w
        pltpu.sync_copy(px_hbm.at[pl.ds(base, n_per_w)], xb)
        pltpu.sync_copy(py_hbm.at[pl.ds(base, n_per_w)], yb)
        pltpu.sync_copy(pz_hbm.at[pl.ds(base, n_per_w)], zb)

        def stage(g, slot):
            idxr = idxs[slot]
            off = g * LANES
            x = jnp.minimum(jnp.maximum(
                (xb[pl.ds(off, LANES)] + 1.0) * 0.5, 0.0), 1.0)
            y = jnp.minimum(jnp.maximum(
                (yb[pl.ds(off, LANES)] + 1.0) * 0.5, 0.0), 1.0)
            z = jnp.minimum(jnp.maximum(
                (zb[pl.ds(off, LANES)] + 1.0) * 0.5, 0.0), 1.0)
            for l in range(L):
                res = float(RES[l])
                px = x * res
                py = y * res
                pz = z * res
                ix = px.astype(jnp.int32)
                iy = py.astype(jnp.int32)
                iz = pz.astype(jnp.int32)
                fx = px - ix.astype(jnp.float32)
                fy = py - iy.astype(jnp.float32)
                fz = pz - iz.astype(jnp.float32)
                hx0 = ix
                hx1 = ix + 1
                hy0 = iy * P1
                hy1 = hy0 + P1
                hz0 = iz * P2
                hz1 = hz0 + P2
                wx1 = fx
                wx0 = 1.0 - fx
                wy1 = fy
                wy0 = 1.0 - fy
                wz1 = fz
                wz0 = 1.0 - fz
                w00 = wx0 * wy0
                w10 = wx1 * wy0
                w01 = wx0 * wy1
                w11 = wx1 * wy1
                lvl_base = l * T
                for c in range(8):
                    hx = hx1 if (c & 1) else hx0
                    hy = hy1 if (c & 2) else hy0
                    hz = hz1 if (c & 4) else hz0
                    h = ((hx ^ hy ^ hz) & HMASK) + lvl_base
                    w0i = h * 2
                    idxr[pl.ds(2 * l * 8 * LANES + c * LANES, LANES)] = w0i
                    idxr[pl.ds((2 * l + 1) * 8 * LANES + c * LANES, LANES)] = w0i + 1
                    wxy = (w11 if (c & 2) else w10) if (c & 1) else (
                        w01 if (c & 2) else w00)
                    w = wxy * (wz1 if (c & 4) else wz0)
                    wb[slot, l, pl.ds(c * LANES, LANES)] = w
            pltpu.async_copy(tab_hbm.at[idxr], gbs[slot], sem)

        def accum(g, slot):
            gbr = gbs[slot]
            pltpu.make_async_copy(
                tab_hbm.at[idxs[slot]], gbr, sem).wait()
            goff = lax.rem(g, GP) * LANES
            for l in range(L):
                acc0 = jnp.zeros((LANES,), jnp.float32)
                acc1 = jnp.zeros((LANES,), jnp.float32)
                for c in range(8):
                    wl = wb[slot, l, pl.ds(c * LANES, LANES)]
                    g0 = gbr[pl.ds(2 * l * 8 * LANES + c * LANES, LANES)]
                    g1 = gbr[pl.ds((2 * l + 1) * 8 * LANES + c * LANES, LANES)]
                    acc0 = acc0 + wl * g0
                    acc1 = acc1 + wl * g1
                ov[2 * l, pl.ds(goff, LANES)] = acc0
                ov[2 * l + 1, pl.ds(goff, LANES)] = acc1

            @pl.when(lax.rem(g, GP) == GP - 1)
            def _flush():
                cbase = (g // GP) * CH
                pltpu.sync_copy(
                    ov, out_hbm.at[pl.ds(0, L * F), pl.ds(base + cbase, CH)])

        stage(0, 0)

        def pair_body(j, _):
            for s in range(2):
                g = 2 * j + s

                @pl.when(g + 1 < n_groups)
                def _():
                    stage(g + 1, 1 - s)

                accum(g, s)
            return 0

        lax.fori_loop(0, n_groups // 2, pair_body, 0)

    return enc(pos_x, pos_y, pos_z, tab)


def _mlp_body(f_ref, d_ref, w0, b0, w1, b1, wcs, wg, bc0, wc1, bc1, wc2, bc2,
              rgb_ref, den_ref):
    f = f_ref[...]
    h = jnp.maximum(
        jnp.dot(w0[...], f, preferred_element_type=jnp.float32) + b0[...], 0.0)
    h2 = jnp.dot(w1[...], h, preferred_element_type=jnp.float32) + b1[...]
    den_ref[...] = jnp.exp(h2[0:1, :] - 1.0)

    d = d_ref[...]
    x = d[0:1, :]
    y = d[1:2, :]
    z = d[2:3, :]
    norm = jnp.sqrt(x * x + y * y + z * z)
    x = x / norm
    y = y / norm
    z = z / norm
    xx = x * x
    yy = y * y
    zz = z * z
    sh = jnp.concatenate([
        0.28209479177387814 * jnp.ones_like(x),
        -0.48860251190291987 * y,
        0.48860251190291987 * z,
        -0.48860251190291987 * x,
        1.0925484305920792 * x * y,
        -1.0925484305920792 * y * z,
        0.94617469575755997 * zz - 0.31539156525252005,
        -1.0925484305920792 * x * z,
        0.54627421529603959 * (xx - yy),
        0.5900435899266435 * y * (3.0 * xx - yy),
        2.890611442640554 * x * y * z,
        0.4570457994644658 * y * (5.0 * zz - 1.0),
        0.3731763325901154 * z * (5.0 * zz - 3.0),
        0.4570457994644658 * x * (5.0 * zz - 1.0),
        1.445305721320277 * z * (xx - yy),
        0.5900435899266435 * x * (xx - 3.0 * yy),
    ], axis=0)
    hc = jnp.maximum(
        jnp.dot(wcs[...], sh, preferred_element_type=jnp.float32)
        + jnp.dot(wg[...], h2, preferred_element_type=jnp.float32)
        + bc0[...], 0.0)
    hc = jnp.maximum(
        jnp.dot(wc1[...], hc, preferred_element_type=jnp.float32) + bc1[...],
        0.0)
    rgb_ref[...] = jax.nn.sigmoid(
        jnp.dot(wc2[...], hc, preferred_element_type=jnp.float32) + bc2[...])


def _mlp(feat_t, dir_t, W0t, b0c, W1t, b1c, Wcst, Wgt, bc0c, Wc1t, bc1c,
         Wc2t, bc2c):
    n = feat_t.shape[1]
    B = 2048
    grid = (n // B,)

    def full(shape):
        return pl.BlockSpec(shape, lambda i: (0, 0))

    return pl.pallas_call(
        _mlp_body,
        grid=grid,
        in_specs=[
            pl.BlockSpec((L * F, B), lambda i: (0, i)),
            pl.BlockSpec((3, B), lambda i: (0, i)),
            full((64, 32)), full((64, 1)),
            full((16, 64)), full((16, 1)),
            full((64, 16)), full((64, 16)), full((64, 1)),
            full((64, 64)), full((64, 1)),
            full((8, 64)), full((8, 1)),
        ],
        out_specs=[
            pl.BlockSpec((8, B), lambda i: (0, i)),
            pl.BlockSpec((1, B), lambda i: (0, i)),
        ],
        out_shape=[
            jax.ShapeDtypeStruct((8, n), jnp.float32),
            jax.ShapeDtypeStruct((1, n), jnp.float32),
        ],
    )(feat_t, dir_t, W0t, b0c, W1t, b1c, Wcst, Wgt, bc0c, Wc1t, bc1c,
      Wc2t, bc2c)


def kernel(positions, directions, table, W0, b0, W1, b1, Wc0, bc0, Wc1, bc1,
           Wc2, bc2):
    n = positions.shape[0]
    tab = table.reshape(L * T * F)
    feat_t = _encode(positions[:, 0], positions[:, 1], positions[:, 2],
                     tab)

    dir_t = directions.T
    Wcst = Wc0[:16].T
    Wgt = jnp.concatenate(
        [jnp.zeros((1, 64), jnp.float32), Wc0[16:]], axis=0).T
    Wc2t = jnp.concatenate(
        [Wc2.T, jnp.zeros((5, 64), jnp.float32)], axis=0)
    bc2c = jnp.concatenate(
        [bc2.reshape(3, 1), jnp.zeros((5, 1), jnp.float32)], axis=0)
    rgb8, den = _mlp(feat_t, dir_t, W0.T, b0.reshape(-1, 1), W1.T,
                     b1.reshape(-1, 1), Wcst, Wgt, bc0.reshape(-1, 1),
                     Wc1.T, bc1.reshape(-1, 1), Wc2t, bc2c)
    return rgb8[:3, :].T, den.reshape(n)


# speedup vs baseline: 4.7051x; 4.7051x over previous
import functools

import numpy as np
import jax
import jax.numpy as jnp
from jax import lax
from jax.experimental import pallas as pl
from jax.experimental.pallas import tpu as pltpu
from jax.experimental.pallas import tpu_sc as plsc

L = 16
F = 2
T = 2 ** 19
BASE = 16
MAXR = 4096
GEO = 15
_b = np.exp((np.log(MAXR) - np.log(BASE)) / (L - 1))
RES = [int(np.floor(BASE * _b ** l)) for l in range(L)]
P1 = int(np.uint32(2654435761).astype(np.int32))
P2 = int(np.uint32(805459861).astype(np.int32))
HMASK = T - 1

NC = 2
NS = 16
LANES = 16
NW = NC * NS

CH = 256
GP = CH // LANES


def _encode(pos_x, pos_y, pos_z, tab):
    n = pos_x.shape[0]
    n_per_w = n // NW
    n_chunks = n_per_w // CH

    n_groups = n_per_w // LANES
    mesh = plsc.VectorSubcoreMesh(core_axis_name="c", subcore_axis_name="s")

    @functools.partial(
        pl.kernel,
        out_type=jax.ShapeDtypeStruct((L * F, n), jnp.float32),
        mesh=mesh,
        scratch_types=[
            pltpu.VMEM((n_per_w,), jnp.float32),
            pltpu.VMEM((n_per_w,), jnp.float32),
            pltpu.VMEM((n_per_w,), jnp.float32),
            pltpu.VMEM((2 * L * 8 * LANES,), jnp.int32),
            pltpu.VMEM((2 * L * 8 * LANES,), jnp.int32),
            pltpu.VMEM((2, L, 8 * LANES), jnp.float32),
            pltpu.VMEM((2 * L * 8 * LANES,), jnp.float32),
            pltpu.VMEM((2 * L * 8 * LANES,), jnp.float32),
            pltpu.VMEM((L * F, CH), jnp.float32),
            pltpu.SemaphoreType.DMA,
        ],
    )
    def enc(px_hbm, py_hbm, pz_hbm, tab_hbm, out_hbm, xb, yb, zb, idx0,
            idx1, wb, gb0, gb1, ov, sem):
        idxs = (idx0, idx1)
        gbs = (gb0, gb1)
        wid = lax.axis_index("s") * NC + lax.axis_index("c")
        base = wid * n_per_w
        pltpu.sync_copy(px_hbm.at[pl.ds(base, n_per_w)], xb)
        pltpu.sync_copy(py_hbm.at[pl.ds(base, n_per_w)], yb)
        pltpu.sync_copy(pz_hbm.at[pl.ds(base, n_per_w)], zb)

        def stage(g, slot):
            idxr = idxs[slot]
            off = g * LANES
            x = jnp.minimum(jnp.maximum(
                (xb[pl.ds(off, LANES)] + 1.0) * 0.5, 0.0), 1.0)
            y = jnp.minimum(jnp.maximum(
                (yb[pl.ds(off, LANES)] + 1.0) * 0.5, 0.0), 1.0)
            z = jnp.minimum(jnp.maximum(
                (zb[pl.ds(off, LANES)] + 1.0) * 0.5, 0.0), 1.0)
            for l in range(L):
                res = float(RES[l])
                px = x * res
                py = y * res
                pz = z * res
                ix = px.astype(jnp.int32)
                iy = py.astype(jnp.int32)
                iz = pz.astype(jnp.int32)
                fx = px - ix.astype(jnp.float32)
                fy = py - iy.astype(jnp.float32)
                fz = pz - iz.astype(jnp.float32)
                hx0 = ix
                hx1 = ix + 1
                hy0 = iy * P1
                hy1 = hy0 + P1
                hz0 = iz * P2
                hz1 = hz0 + P2
                wx1 = fx
                wx0 = 1.0 - fx
                wy1 = fy
                wy0 = 1.0 - fy
                wz1 = fz
                wz0 = 1.0 - fz
                w00 = wx0 * wy0
                w10 = wx1 * wy0
                w01 = wx0 * wy1
                w11 = wx1 * wy1
                lvl_base = l * T * F
                for c in range(8):
                    hx = hx1 if (c & 1) else hx0
                    hy = hy1 if (c & 2) else hy0
                    hz = hz1 if (c & 4) else hz0
                    h = (hx ^ hy ^ hz) & HMASK
                    h2v = h * 2
                    w0i = (h2v & ~255) + (h & 127) + lvl_base
                    idxr[pl.ds(2 * l * 8 * LANES + c * LANES, LANES)] = w0i
                    idxr[pl.ds((2 * l + 1) * 8 * LANES + c * LANES, LANES)] = (
                        w0i)
                    wxy = (w11 if (c & 2) else w10) if (c & 1) else (
                        w01 if (c & 2) else w00)
                    w = wxy * (wz1 if (c & 4) else wz0)
                    wb[slot, l, pl.ds(c * LANES, LANES)] = w
            pltpu.async_copy(tab_hbm.at[idxr], gbs[slot], sem)

        def accum(g, slot):
            gbr = gbs[slot]
            pltpu.make_async_copy(
                tab_hbm.at[idxs[slot]], gbr, sem).wait()
            goff = lax.rem(g, GP) * LANES
            for l in range(L):
                acc0 = jnp.zeros((LANES,), jnp.float32)
                acc1 = jnp.zeros((LANES,), jnp.float32)
                for c in range(8):
                    wl = wb[slot, l, pl.ds(c * LANES, LANES)]
                    g0 = gbr[pl.ds(2 * l * 8 * LANES + c * LANES, LANES)]
                    g1 = gbr[pl.ds((2 * l + 1) * 8 * LANES + c * LANES, LANES)]
                    acc0 = acc0 + wl * g0
                    acc1 = acc1 + wl * g1
                ov[2 * l, pl.ds(goff, LANES)] = acc0
                ov[2 * l + 1, pl.ds(goff, LANES)] = acc1

            @pl.when(lax.rem(g, GP) == GP - 1)
            def _flush():
                cbase = (g // GP) * CH
                pltpu.sync_copy(
                    ov, out_hbm.at[pl.ds(0, L * F), pl.ds(base + cbase, CH)])

        stage(0, 0)

        def pair_body(j, _):
            for s in range(2):
                g = 2 * j + s

                @pl.when(g + 1 < n_groups)
                def _():
                    stage(g + 1, 1 - s)

                accum(g, s)
            return 0

        lax.fori_loop(0, n_groups // 2, pair_body, 0)

    return enc(pos_x, pos_y, pos_z, tab)


def _mlp_body(f_ref, d_ref, w0, b0, w1, b1, wcs, wg, bc0, wc1, bc1, wc2, bc2,
              rgb_ref, den_ref):
    f = f_ref[...]
    h = jnp.maximum(
        jnp.dot(w0[...], f, preferred_element_type=jnp.float32) + b0[...], 0.0)
    h2 = jnp.dot(w1[...], h, preferred_element_type=jnp.float32) + b1[...]
    den_ref[...] = jnp.exp(h2[0:1, :] - 1.0)

    d = d_ref[...]
    x = d[0:1, :]
    y = d[1:2, :]
    z = d[2:3, :]
    norm = jnp.sqrt(x * x + y * y + z * z)
    x = x / norm
    y = y / norm
    z = z / norm
    xx = x * x
    yy = y * y
    zz = z * z
    sh = jnp.concatenate([
        0.28209479177387814 * jnp.ones_like(x),
        -0.48860251190291987 * y,
        0.48860251190291987 * z,
        -0.48860251190291987 * x,
        1.0925484305920792 * x * y,
        -1.0925484305920792 * y * z,
        0.94617469575755997 * zz - 0.31539156525252005,
        -1.0925484305920792 * x * z,
        0.54627421529603959 * (xx - yy),
        0.5900435899266435 * y * (3.0 * xx - yy),
        2.890611442640554 * x * y * z,
        0.4570457994644658 * y * (5.0 * zz - 1.0),
        0.3731763325901154 * z * (5.0 * zz - 3.0),
        0.4570457994644658 * x * (5.0 * zz - 1.0),
        1.445305721320277 * z * (xx - yy),
        0.5900435899266435 * x * (xx - 3.0 * yy),
    ], axis=0)
    hc = jnp.maximum(
        jnp.dot(wcs[...], sh, preferred_element_type=jnp.float32)
        + jnp.dot(wg[...], h2, preferred_element_type=jnp.float32)
        + bc0[...], 0.0)
    hc = jnp.maximum(
        jnp.dot(wc1[...], hc, preferred_element_type=jnp.float32) + bc1[...],
        0.0)
    rgb_ref[...] = jax.nn.sigmoid(
        jnp.dot(wc2[...], hc, preferred_element_type=jnp.float32) + bc2[...])


def _mlp(feat_t, dir_t, W0t, b0c, W1t, b1c, Wcst, Wgt, bc0c, Wc1t, bc1c,
         Wc2t, bc2c):
    n = feat_t.shape[1]
    B = 2048
    grid = (n // B,)

    def full(shape):
        return pl.BlockSpec(shape, lambda i: (0, 0))

    return pl.pallas_call(
        _mlp_body,
        grid=grid,
        in_specs=[
            pl.BlockSpec((L * F, B), lambda i: (0, i)),
            pl.BlockSpec((3, B), lambda i: (0, i)),
            full((64, 32)), full((64, 1)),
            full((16, 64)), full((16, 1)),
            full((64, 16)), full((64, 16)), full((64, 1)),
            full((64, 64)), full((64, 1)),
            full((8, 64)), full((8, 1)),
        ],
        out_specs=[
            pl.BlockSpec((8, B), lambda i: (0, i)),
            pl.BlockSpec((1, B), lambda i: (0, i)),
        ],
        out_shape=[
            jax.ShapeDtypeStruct((8, n), jnp.float32),
            jax.ShapeDtypeStruct((1, n), jnp.float32),
        ],
    )(feat_t, dir_t, W0t, b0c, W1t, b1c, Wcst, Wgt, bc0c, Wc1t, bc1c,
      Wc2t, bc2c)


def kernel(positions, directions, table, W0, b0, W1, b1, Wc0, bc0, Wc1, bc1,
           Wc2, bc2):
    n = positions.shape[0]
    tab = table.reshape(L, T // 128, 128, F).transpose(0, 1, 3, 2).reshape(
        L * T * F)
    feat_t = _encode(positions[:, 0], positions[:, 1], positions[:, 2],
                     tab)

    dir_t = directions.T
    Wcst = Wc0[:16].T
    Wgt = jnp.concatenate(
        [jnp.zeros((1, 64), jnp.float32), Wc0[16:]], axis=0).T
    Wc2t = jnp.concatenate(
        [Wc2.T, jnp.zeros((5, 64), jnp.float32)], axis=0)
    bc2c = jnp.concatenate(
        [bc2.reshape(3, 1), jnp.zeros((5, 1), jnp.float32)], axis=0)
    rgb8, den = _mlp(feat_t, dir_t, W0.T, b0.reshape(-1, 1), W1.T,
                     b1.reshape(-1, 1), Wcst, Wgt, bc0.reshape(-1, 1),
                     Wc1.T, bc1.reshape(-1, 1), Wc2t, bc2c)
    return rgb8[:3, :].T, den.reshape(n)


# speedup vs baseline: 4.9023x; 1.0419x over previous
import functools

import numpy as np
import jax
import jax.numpy as jnp
from jax import lax
from jax.experimental import pallas as pl
from jax.experimental.pallas import tpu as pltpu
from jax.experimental.pallas import tpu_sc as plsc

L = 16
F = 2
T = 2 ** 19
BASE = 16
MAXR = 4096
GEO = 15
_b = np.exp((np.log(MAXR) - np.log(BASE)) / (L - 1))
RES = [int(np.floor(BASE * _b ** l)) for l in range(L)]
P1 = int(np.uint32(2654435761).astype(np.int32))
P2 = int(np.uint32(805459861).astype(np.int32))
HMASK = T - 1

NC = 2
NS = 16
LANES = 16
NW = NC * NS

CH = 256
GP = CH // LANES


def _encode(pos_x, pos_y, pos_z, tab):
    n = pos_x.shape[0]
    n_per_w = n // NW
    n_chunks = n_per_w // CH

    n_groups = n_per_w // LANES
    mesh = plsc.VectorSubcoreMesh(core_axis_name="c", subcore_axis_name="s")

    @functools.partial(
        pl.kernel,
        out_type=jax.ShapeDtypeStruct((L * F, n), jnp.float32),
        mesh=mesh,
        scratch_types=[
            pltpu.VMEM((n_per_w,), jnp.float32),
            pltpu.VMEM((n_per_w,), jnp.float32),
            pltpu.VMEM((n_per_w,), jnp.float32),
            pltpu.VMEM((2 * L * 8 * LANES,), jnp.int32),
            pltpu.VMEM((2 * L * 8 * LANES,), jnp.int32),
            pltpu.VMEM((2, L, 8 * LANES), jnp.float32),
            pltpu.VMEM((2 * L * 8 * LANES,), jnp.float32),
            pltpu.VMEM((2 * L * 8 * LANES,), jnp.float32),
            pltpu.VMEM((L * F, CH), jnp.float32),
            pltpu.SemaphoreType.DMA,
        ],
    )
    def enc(px_hbm, py_hbm, pz_hbm, tab_hbm, out_hbm, xb, yb, zb, idx0,
            idx1, wb, gb0, gb1, ov, sem):
        idxs = (idx0, idx1)
        gbs = (gb0, gb1)
        wid = lax.axis_index("s") * NC + lax.axis_index("c")
        base = wid * n_per_w
        pltpu.sync_copy(px_hbm.at[pl.ds(base, n_per_w)], xb)
        pltpu.sync_copy(py_hbm.at[pl.ds(base, n_per_w)], yb)
        pltpu.sync_copy(pz_hbm.at[pl.ds(base, n_per_w)], zb)

        def stage(g, slot):
            idxr = idxs[slot]
            off = g * LANES
            x = jnp.minimum(jnp.maximum(
                (xb[pl.ds(off, LANES)] + 1.0) * 0.5, 0.0), 1.0)
            y = jnp.minimum(jnp.maximum(
                (yb[pl.ds(off, LANES)] + 1.0) * 0.5, 0.0), 1.0)
            z = jnp.minimum(jnp.maximum(
                (zb[pl.ds(off, LANES)] + 1.0) * 0.5, 0.0), 1.0)
            for l in range(L):
                res = float(RES[l])
                px = x * res
                py = y * res
                pz = z * res
                ix = px.astype(jnp.int32)
                iy = py.astype(jnp.int32)
                iz = pz.astype(jnp.int32)
                fx = px - ix.astype(jnp.float32)
                fy = py - iy.astype(jnp.float32)
                fz = pz - iz.astype(jnp.float32)
                hx0 = ix
                hx1 = ix + 1
                hy0 = iy * P1
                hy1 = hy0 + P1
                hz0 = iz * P2
                hz1 = hz0 + P2
                wx1 = fx
                wx0 = 1.0 - fx
                wy1 = fy
                wy0 = 1.0 - fy
                wz1 = fz
                wz0 = 1.0 - fz
                w00 = wx0 * wy0
                w10 = wx1 * wy0
                w01 = wx0 * wy1
                w11 = wx1 * wy1
                lvl_base = l * T * F
                for c in range(8):
                    hx = hx1 if (c & 1) else hx0
                    hy = hy1 if (c & 2) else hy0
                    hz = hz1 if (c & 4) else hz0
                    h = (hx ^ hy ^ hz) & HMASK
                    h2v = h * 2
                    w0i = (h2v & ~255) + (h & 127) + lvl_base
                    idxr[pl.ds(2 * l * 8 * LANES + c * LANES, LANES)] = w0i
                    idxr[pl.ds((2 * l + 1) * 8 * LANES + c * LANES, LANES)] = (
                        w0i + 128)
                    wxy = (w11 if (c & 2) else w10) if (c & 1) else (
                        w01 if (c & 2) else w00)
                    w = wxy * (wz1 if (c & 4) else wz0)
                    wb[slot, l, pl.ds(c * LANES, LANES)] = w
            pltpu.async_copy(tab_hbm.at[idxr], gbs[slot], sem)

        def accum(g, slot):
            gbr = gbs[slot]
            pltpu.make_async_copy(
                tab_hbm.at[idxs[slot]], gbr, sem).wait()
            goff = lax.rem(g, GP) * LANES
            for l in range(L):
                acc0 = jnp.zeros((LANES,), jnp.float32)
                acc1 = jnp.zeros((LANES,), jnp.float32)
                for c in range(8):
                    wl = wb[slot, l, pl.ds(c * LANES, LANES)]
                    g0 = gbr[pl.ds(2 * l * 8 * LANES + c * LANES, LANES)]
                    g1 = gbr[pl.ds((2 * l + 1) * 8 * LANES + c * LANES, LANES)]
                    acc0 = acc0 + wl * g0
                    acc1 = acc1 + wl * g1
                ov[2 * l, pl.ds(goff, LANES)] = acc0
                ov[2 * l + 1, pl.ds(goff, LANES)] = acc1

            @pl.when(lax.rem(g, GP) == GP - 1)
            def _flush():
                cbase = (g // GP) * CH
                pltpu.sync_copy(
                    ov, out_hbm.at[pl.ds(0, L * F), pl.ds(base + cbase, CH)])

        stage(0, 0)

        def pair_body(j, _):
            for s in range(2):
                g = 2 * j + s

                @pl.when(g + 1 < n_groups)
                def _():
                    stage(g + 1, 1 - s)

                accum(g, s)
            return 0

        lax.fori_loop(0, n_groups // 2, pair_body, 0)

    return enc(pos_x, pos_y, pos_z, tab)


def _mlp_body(f_ref, d_ref, w0, b0, w1, b1, wcs, wg, bc0, wc1, bc1, wc2, bc2,
              rgb_ref, den_ref):
    f = f_ref[...]
    h = jnp.maximum(
        jnp.dot(w0[...], f, preferred_element_type=jnp.float32) + b0[...], 0.0)
    h2 = jnp.dot(w1[...], h, preferred_element_type=jnp.float32) + b1[...]
    den_ref[...] = jnp.exp(h2[0:1, :] - 1.0)

    d = d_ref[...]
    x = d[0:1, :]
    y = d[1:2, :]
    z = d[2:3, :]
    norm = jnp.sqrt(x * x + y * y + z * z)
    x = x / norm
    y = y / norm
    z = z / norm
    xx = x * x
    yy = y * y
    zz = z * z
    sh = jnp.concatenate([
        0.28209479177387814 * jnp.ones_like(x),
        -0.48860251190291987 * y,
        0.48860251190291987 * z,
        -0.48860251190291987 * x,
        1.0925484305920792 * x * y,
        -1.0925484305920792 * y * z,
        0.94617469575755997 * zz - 0.31539156525252005,
        -1.0925484305920792 * x * z,
        0.54627421529603959 * (xx - yy),
        0.5900435899266435 * y * (3.0 * xx - yy),
        2.890611442640554 * x * y * z,
        0.4570457994644658 * y * (5.0 * zz - 1.0),
        0.3731763325901154 * z * (5.0 * zz - 3.0),
        0.4570457994644658 * x * (5.0 * zz - 1.0),
        1.445305721320277 * z * (xx - yy),
        0.5900435899266435 * x * (xx - 3.0 * yy),
    ], axis=0)
    hc = jnp.maximum(
        jnp.dot(wcs[...], sh, preferred_element_type=jnp.float32)
        + jnp.dot(wg[...], h2, preferred_element_type=jnp.float32)
        + bc0[...], 0.0)
    hc = jnp.maximum(
        jnp.dot(wc1[...], hc, preferred_element_type=jnp.float32) + bc1[...],
        0.0)
    rgb_ref[...] = jax.nn.sigmoid(
        jnp.dot(wc2[...], hc, preferred_element_type=jnp.float32) + bc2[...])


def _mlp(feat_t, dir_t, W0t, b0c, W1t, b1c, Wcst, Wgt, bc0c, Wc1t, bc1c,
         Wc2t, bc2c):
    n = feat_t.shape[1]
    B = 2048
    grid = (n // B,)

    def full(shape):
        return pl.BlockSpec(shape, lambda i: (0, 0))

    return pl.pallas_call(
        _mlp_body,
        grid=grid,
        in_specs=[
            pl.BlockSpec((L * F, B), lambda i: (0, i)),
            pl.BlockSpec((3, B), lambda i: (0, i)),
            full((64, 32)), full((64, 1)),
            full((16, 64)), full((16, 1)),
            full((64, 16)), full((64, 16)), full((64, 1)),
            full((64, 64)), full((64, 1)),
            full((8, 64)), full((8, 1)),
        ],
        out_specs=[
            pl.BlockSpec((8, B), lambda i: (0, i)),
            pl.BlockSpec((1, B), lambda i: (0, i)),
        ],
        out_shape=[
            jax.ShapeDtypeStruct((8, n), jnp.float32),
            jax.ShapeDtypeStruct((1, n), jnp.float32),
        ],
    )(feat_t, dir_t, W0t, b0c, W1t, b1c, Wcst, Wgt, bc0c, Wc1t, bc1c,
      Wc2t, bc2c)


def kernel(positions, directions, table, W0, b0, W1, b1, Wc0, bc0, Wc1, bc1,
           Wc2, bc2):
    n = positions.shape[0]
    tab = table.reshape(L, T // 128, 128, F).transpose(0, 1, 3, 2).reshape(
        L * T * F)
    feat_t = _encode(positions[:, 0], positions[:, 1], positions[:, 2],
                     tab)

    dir_t = directions.T
    Wcst = Wc0[:16].T
    Wgt = jnp.concatenate(
        [jnp.zeros((1, 64), jnp.float32), Wc0[16:]], axis=0).T
    Wc2t = jnp.concatenate(
        [Wc2.T, jnp.zeros((5, 64), jnp.float32)], axis=0)
    bc2c = jnp.concatenate(
        [bc2.reshape(3, 1), jnp.zeros((5, 1), jnp.float32)], axis=0)
    rgb8, den = _mlp(feat_t, dir_t, W0.T, b0.reshape(-1, 1), W1.T,
                     b1.reshape(-1, 1), Wcst, Wgt, bc0.reshape(-1, 1),
                     Wc1.T, bc1.reshape(-1, 1), Wc2t, bc2c)
    return rgb8[:3, :].T, den.reshape(n)
